# baseline (speedup 1.0000x reference)
import functools

import jax
import jax.numpy as jnp
from jax import lax
from jax.experimental import pallas as pl
from jax.experimental.pallas import tpu as pltpu
from jax.experimental.pallas import tpu_sc as plsc

NC = 2
NS = 16
NW = NC * NS
BLK = 128
NIB = 4
LANES = 16


def _pad_to(arr, n, fill):
    pad = n - arr.shape[0]
    if pad == 0:
        return arr
    return jnp.concatenate([arr, jnp.full((pad,), fill, arr.dtype)], axis=0)


def _sc_gather_scale_scatter(x, pair, w3, n_nodes):
    V, D = x.shape
    nblk = pair.shape[1]
    rows_per_sub = n_nodes // NS
    mesh = plsc.VectorSubcoreMesh(core_axis_name="c", subcore_axis_name="s")

    @functools.partial(
        pl.kernel,
        mesh=mesh,
        out_type=jax.ShapeDtypeStruct((NC, n_nodes, D), x.dtype),
        scratch_types=[
            pltpu.VMEM((NIB, 2, BLK), jnp.int32),
            pltpu.VMEM((NIB, 1, BLK), jnp.float32),
            pltpu.VMEM((2, BLK, D), x.dtype),
            pltpu.VMEM_SHARED((n_nodes, D), x.dtype),
            pltpu.SemaphoreType.DMA((NIB,)),
            pltpu.SemaphoreType.DMA((2,)),
            pltpu.SemaphoreType.DMA((2,)),
        ],
    )
    def k(x_hbm, pair_hbm, w_hbm, out_hbm, pair_v, w_v, rows_v, agg_sh,
          isem, gsem, ssem):
        cid = lax.axis_index("c")
        sid = lax.axis_index("s")
        wid = sid * NC + cid

        def stage_idx(blk_i, slot):
            pltpu.async_copy(pair_hbm.at[wid, blk_i], pair_v.at[slot],
                             isem.at[slot])
            pltpu.async_copy(w_hbm.at[wid, blk_i], w_v.at[slot],
                             isem.at[slot])

        def wait_idx(slot):
            pltpu.make_async_copy(pair_hbm.at[wid, 0], pair_v.at[slot],
                                  isem.at[slot]).wait()
            pltpu.make_async_copy(w_hbm.at[wid, 0], w_v.at[slot],
                                  isem.at[slot]).wait()

        def start_gather(slot, rb):
            pltpu.async_copy(x_hbm.at[pair_v.at[slot, 0]], rows_v.at[rb],
                             gsem.at[rb])

        def wait_gather(rb):
            pltpu.make_async_copy(x_hbm.at[pair_v.at[0, 0]], rows_v.at[rb],
                                  gsem.at[rb]).wait()

        def start_scatter(slot, rb):
            pltpu.async_copy(rows_v.at[rb], agg_sh.at[pair_v.at[slot, 1]],
                             ssem.at[rb], add=True)

        def wait_scatter(rb):
            pltpu.make_async_copy(rows_v.at[rb], agg_sh.at[pair_v.at[0, 1]],
                                  ssem.at[rb]).wait()

        for slot in range(NIB):
            stage_idx(slot, slot)

        @pl.loop(0, BLK)
        def _(i):
            @pl.loop(0, D // LANES)
            def _(j):
                rows_v[0, i, pl.ds(j * LANES, LANES)] = jnp.zeros(
                    (LANES,), x.dtype)

        @pl.loop(0, rows_per_sub // BLK)
        def _(i):
            r0 = sid * rows_per_sub + i * BLK
            pltpu.sync_copy(rows_v.at[0], agg_sh.at[pl.ds(r0, BLK), :])

        plsc.subcore_barrier()

        wait_idx(0)
        start_gather(0, 0)

        @pl.loop(0, nblk, step=4)
        def _(s):
            for b in range(4):
                j = s + b
                rb = b % 2

                @pl.when(j + 1 < nblk)
                def _():
                    @pl.when(j >= 1)
                    def _():
                        wait_scatter(1 - rb)

                        @pl.when(j + 3 < nblk)
                        def _():
                            stage_idx(j + 3, (b + 3) % 4)

                    wait_idx((b + 1) % 4)
                    start_gather((b + 1) % 4, 1 - rb)

                wait_gather(rb)

                @pl.loop(0, BLK // LANES)
                def _(g):
                    wv = w_v[b, 0, pl.ds(g * LANES, LANES)]
                    for l in range(LANES):
                        w_e = wv[l]
                        e = g * LANES + l
                        for c in range(D // LANES):
                            sl = pl.ds(c * LANES, LANES)
                            rows_v[rb, e, sl] = rows_v[rb, e, sl] * w_e

                start_scatter(b, rb)

        wait_scatter(0)
        wait_scatter(1)
        plsc.subcore_barrier()

        r0 = sid * rows_per_sub
        pltpu.sync_copy(agg_sh.at[pl.ds(r0, rows_per_sub), :],
                        out_hbm.at[cid, pl.ds(r0, rows_per_sub), :])

    return k(x, pair, w3)


def _sc_pair_dot(h, pair):
    V, D = h.shape
    nblk = pair.shape[1]
    B = NW * nblk * BLK
    mesh = plsc.VectorSubcoreMesh(core_axis_name="c", subcore_axis_name="s")

    @functools.partial(
        pl.kernel,
        mesh=mesh,
        out_type=jax.ShapeDtypeStruct((B, LANES), h.dtype),
        scratch_types=[
            pltpu.VMEM((NIB, 2, BLK), jnp.int32),
            pltpu.VMEM((2, BLK, D), h.dtype),
            pltpu.VMEM((2, BLK, D), h.dtype),
            pltpu.VMEM((2, BLK, LANES), h.dtype),
            pltpu.SemaphoreType.DMA((NIB,)),
            pltpu.SemaphoreType.DMA((2,)),
            pltpu.SemaphoreType.DMA((2,)),
            pltpu.SemaphoreType.DMA((2,)),
        ],
    )
    def k(h_hbm, pair_hbm, out_hbm, pair_v, ra_v, rb_v, o_v,
          isem, asem, bsem, wsem):
        wid = lax.axis_index("s") * NC + lax.axis_index("c")
        base_row = wid * nblk * BLK

        def stage_idx(blk_i, slot):
            pltpu.async_copy(pair_hbm.at[wid, blk_i], pair_v.at[slot],
                             isem.at[slot])

        def wait_idx(slot):
            pltpu.make_async_copy(pair_hbm.at[wid, 0], pair_v.at[slot],
                                  isem.at[slot]).wait()

        def start_gathers(slot, rb):
            pltpu.async_copy(h_hbm.at[pair_v.at[slot, 0]], ra_v.at[rb],
                             asem.at[rb])
            pltpu.async_copy(h_hbm.at[pair_v.at[slot, 1]], rb_v.at[rb],
                             bsem.at[rb])

        def wait_gathers(rb):
            pltpu.make_async_copy(h_hbm.at[pair_v.at[0, 0]], ra_v.at[rb],
                                  asem.at[rb]).wait()
            pltpu.make_async_copy(h_hbm.at[pair_v.at[0, 1]], rb_v.at[rb],
                                  bsem.at[rb]).wait()

        def start_writeout(blk_i, rb):
            off = base_row + blk_i * BLK
            pltpu.async_copy(o_v.at[rb], out_hbm.at[pl.ds(off, BLK), :],
                             wsem.at[rb])

        def wait_writeout(rb):
            pltpu.make_async_copy(o_v.at[rb], out_hbm.at[pl.ds(0, BLK), :],
                                  wsem.at[rb]).wait()

        for slot in range(NIB):
            stage_idx(slot, slot)
        wait_idx(0)
        start_gathers(0, 0)

        @pl.loop(0, nblk, step=4)
        def _(s):
            for b in range(4):
                j = s + b
                rb = b % 2

                @pl.when(j + 1 < nblk)
                def _():
                    wait_idx((b + 1) % 4)
                    start_gathers((b + 1) % 4, 1 - rb)

                wait_gathers(rb)

                @pl.when(j + 4 < nblk)
                def _():
                    stage_idx(j + 4, b)

                @pl.when(j >= 2)
                def _():
                    wait_writeout(rb)

                @pl.loop(0, BLK)
                def _(e):
                    acc = (ra_v[rb, e, pl.ds(0, LANES)] *
                           rb_v[rb, e, pl.ds(0, LANES)])
                    for c in range(1, D // LANES):
                        sl = pl.ds(c * LANES, LANES)
                        acc = acc + ra_v[rb, e, sl] * rb_v[rb, e, sl]
                    o_v[rb, e, :] = acc

                start_writeout(j, rb)

        wait_writeout(0)
        wait_writeout(1)

    return k(h, pair)


def _tc_linear(partials, W, b_row):
    _, N, D = partials.shape
    blk = 2048

    def body(p_ref, w_ref, b_ref, o_ref):
        s = p_ref[0] + p_ref[1]
        o_ref[...] = jnp.dot(s, w_ref[...],
                             preferred_element_type=jnp.float32) + b_ref[...]

    return pl.pallas_call(
        body,
        grid=(N // blk,),
        in_specs=[pl.BlockSpec((NC, blk, D), lambda i: (0, i, 0)),
                  pl.BlockSpec((D, D), lambda i: (0, 0)),
                  pl.BlockSpec((1, D), lambda i: (0, 0))],
        out_specs=pl.BlockSpec((blk, D), lambda i: (i, 0)),
        out_shape=jax.ShapeDtypeStruct((N, D), jnp.float32),
    )(partials, W, b_row)


def _tc_lanesum(parts):
    B, L = parts.shape
    blk = 4096

    def body(p_ref, o_ref):
        ones = jnp.ones((L, 1), jnp.float32)
        o_ref[...] = jnp.dot(p_ref[...], ones,
                             preferred_element_type=jnp.float32)

    return pl.pallas_call(
        body,
        grid=(B // blk,),
        in_specs=[pl.BlockSpec((blk, L), lambda i: (i, 0))],
        out_specs=pl.BlockSpec((blk, 1), lambda i: (i, 0)),
        out_shape=jax.ShapeDtypeStruct((B, 1), jnp.float32),
    )(parts)


def kernel(x, edge_index, edge_weight, edges, W, b):
    n_nodes, d = x.shape
    n_edges = edge_weight.shape[0]
    n_cand = edges.shape[1]

    unit = NW * BLK * 4
    e_pad = ((n_edges + unit - 1) // unit) * unit
    c_pad = ((n_cand + unit - 1) // unit) * unit
    nblk_e = e_pad // (NW * BLK)
    nblk_c = c_pad // (NW * BLK)

    src = _pad_to(edge_index[0].astype(jnp.int32), e_pad, 0)
    dst = _pad_to(edge_index[1].astype(jnp.int32), e_pad, 0)
    wts = _pad_to(edge_weight, e_pad, 0.0)
    e0 = _pad_to(edges[0].astype(jnp.int32), c_pad, 0)
    e1 = _pad_to(edges[1].astype(jnp.int32), c_pad, 0)

    epair = jnp.stack([src.reshape(NW, nblk_e, BLK),
                       dst.reshape(NW, nblk_e, BLK)], axis=2)
    w3 = wts.reshape(NW, nblk_e, 1, BLK)
    cpair = jnp.stack([e0.reshape(NW, nblk_c, BLK),
                       e1.reshape(NW, nblk_c, BLK)], axis=2)

    n_pad = ((n_nodes + NS * BLK - 1) // (NS * BLK)) * (NS * BLK)

    partials = _sc_gather_scale_scatter(x, epair, w3, n_pad)
    h = _tc_linear(partials, W, b.reshape(1, d))
    parts = _sc_pair_dot(h, cpair)
    out = _tc_lanesum(parts)
    return out[:n_cand, 0]


# speedup vs baseline: 1.0010x; 1.0010x over previous
import functools

import jax
import jax.numpy as jnp
from jax import lax
from jax.experimental import pallas as pl
from jax.experimental.pallas import tpu as pltpu
from jax.experimental.pallas import tpu_sc as plsc

NC = 2
NS = 16
NW = NC * NS
BLK = 128
NIB = 4
LANES = 16


def _pad_to(arr, n, fill):
    pad = n - arr.shape[0]
    if pad == 0:
        return arr
    return jnp.concatenate([arr, jnp.full((pad,), fill, arr.dtype)], axis=0)


def _sc_gather_scale_scatter(x, pair, w3, n_nodes):
    V, D = x.shape
    nblk = pair.shape[1]
    rows_per_sub = n_nodes // NS
    mesh = plsc.VectorSubcoreMesh(core_axis_name="c", subcore_axis_name="s")

    @functools.partial(
        pl.kernel,
        mesh=mesh,
        out_type=jax.ShapeDtypeStruct((NC, n_nodes, D), x.dtype),
        scratch_types=[
            pltpu.VMEM((NIB, 2, BLK), jnp.int32),
            pltpu.VMEM((NIB, 1, BLK), jnp.float32),
            pltpu.VMEM((2, BLK, D), x.dtype),
            pltpu.VMEM_SHARED((n_nodes, D), x.dtype),
            pltpu.SemaphoreType.DMA((NIB,)),
            pltpu.SemaphoreType.DMA((2,)),
            pltpu.SemaphoreType.DMA((2,)),
        ],
    )
    def k(x_hbm, pair_hbm, w_hbm, out_hbm, pair_v, w_v, rows_v, agg_sh,
          isem, gsem, ssem):
        cid = lax.axis_index("c")
        sid = lax.axis_index("s")
        wid = sid * NC + cid

        def stage_idx(blk_i, slot):
            pltpu.async_copy(pair_hbm.at[wid, blk_i], pair_v.at[slot],
                             isem.at[slot])
            pltpu.async_copy(w_hbm.at[wid, blk_i], w_v.at[slot],
                             isem.at[slot])

        def wait_idx(slot):
            pltpu.make_async_copy(pair_hbm.at[wid, 0], pair_v.at[slot],
                                  isem.at[slot]).wait()
            pltpu.make_async_copy(w_hbm.at[wid, 0], w_v.at[slot],
                                  isem.at[slot]).wait()

        def start_gather(slot, rb):
            pltpu.async_copy(x_hbm.at[pair_v.at[slot, 0]], rows_v.at[rb],
                             gsem.at[rb])

        def wait_gather(rb):
            pltpu.make_async_copy(x_hbm.at[pair_v.at[0, 0]], rows_v.at[rb],
                                  gsem.at[rb]).wait()

        def start_scatter(slot, rb):
            pltpu.async_copy(rows_v.at[rb], agg_sh.at[pair_v.at[slot, 1]],
                             ssem.at[rb], add=True)

        def wait_scatter(rb):
            pltpu.make_async_copy(rows_v.at[rb], agg_sh.at[pair_v.at[0, 1]],
                                  ssem.at[rb]).wait()

        for slot in range(NIB):
            stage_idx(slot, slot)

        @pl.loop(0, BLK)
        def _(i):
            @pl.loop(0, D // LANES)
            def _(j):
                rows_v[0, i, pl.ds(j * LANES, LANES)] = jnp.zeros(
                    (LANES,), x.dtype)

        @pl.loop(0, rows_per_sub // BLK)
        def _(i):
            r0 = sid * rows_per_sub + i * BLK
            pltpu.sync_copy(rows_v.at[0], agg_sh.at[pl.ds(r0, BLK), :])

        plsc.subcore_barrier()

        wait_idx(0)
        start_gather(0, 0)

        @pl.loop(0, nblk, step=4)
        def _(s):
            for b in range(4):
                j = s + b
                rb = b % 2

                @pl.when(j + 1 < nblk)
                def _():
                    @pl.when(j >= 1)
                    def _():
                        wait_scatter(1 - rb)

                        @pl.when(j + 3 < nblk)
                        def _():
                            stage_idx(j + 3, (b + 3) % 4)

                    wait_idx((b + 1) % 4)
                    start_gather((b + 1) % 4, 1 - rb)

                wait_gather(rb)

                @pl.loop(0, BLK // LANES)
                def _(g):
                    wv = w_v[b, 0, pl.ds(g * LANES, LANES)]
                    for l in range(LANES):
                        w_e = wv[l]
                        e = g * LANES + l
                        for c in range(D // LANES):
                            sl = pl.ds(c * LANES, LANES)
                            rows_v[rb, e, sl] = rows_v[rb, e, sl] * w_e

                start_scatter(b, rb)

        wait_scatter(0)
        wait_scatter(1)
        plsc.subcore_barrier()

        r0 = sid * rows_per_sub
        pltpu.sync_copy(agg_sh.at[pl.ds(r0, rows_per_sub), :],
                        out_hbm.at[cid, pl.ds(r0, rows_per_sub), :])

    return k(x, pair, w3)


def _sc_pair_dot(h, pair):
    V, D = h.shape
    nblk = pair.shape[1]
    B = NW * nblk * BLK
    mesh = plsc.VectorSubcoreMesh(core_axis_name="c", subcore_axis_name="s")

    @functools.partial(
        pl.kernel,
        mesh=mesh,
        out_type=jax.ShapeDtypeStruct((B, LANES), h.dtype),
        scratch_types=[
            pltpu.VMEM((NIB, 2, BLK), jnp.int32),
            pltpu.VMEM((2, BLK, D), h.dtype),
            pltpu.VMEM((2, BLK, D), h.dtype),
            pltpu.VMEM((2, BLK, LANES), h.dtype),
            pltpu.SemaphoreType.DMA((NIB,)),
            pltpu.SemaphoreType.DMA((2,)),
            pltpu.SemaphoreType.DMA((2,)),
            pltpu.SemaphoreType.DMA((2,)),
        ],
    )
    def k(h_hbm, pair_hbm, out_hbm, pair_v, ra_v, rb_v, o_v,
          isem, asem, bsem, wsem):
        wid = lax.axis_index("s") * NC + lax.axis_index("c")
        base_row = wid * nblk * BLK

        def stage_idx(blk_i, slot):
            pltpu.async_copy(pair_hbm.at[wid, blk_i], pair_v.at[slot],
                             isem.at[slot])

        def wait_idx(slot):
            pltpu.make_async_copy(pair_hbm.at[wid, 0], pair_v.at[slot],
                                  isem.at[slot]).wait()

        def start_gathers(slot, rb):
            pltpu.async_copy(h_hbm.at[pair_v.at[slot, 0]], ra_v.at[rb],
                             asem.at[rb])
            pltpu.async_copy(h_hbm.at[pair_v.at[slot, 1]], rb_v.at[rb],
                             bsem.at[rb])

        def wait_gathers(rb):
            pltpu.make_async_copy(h_hbm.at[pair_v.at[0, 0]], ra_v.at[rb],
                                  asem.at[rb]).wait()
            pltpu.make_async_copy(h_hbm.at[pair_v.at[0, 1]], rb_v.at[rb],
                                  bsem.at[rb]).wait()

        def start_writeout(blk_i, rb):
            off = base_row + blk_i * BLK
            pltpu.async_copy(o_v.at[rb], out_hbm.at[pl.ds(off, BLK), :],
                             wsem.at[rb])

        def wait_writeout(rb):
            pltpu.make_async_copy(o_v.at[rb], out_hbm.at[pl.ds(0, BLK), :],
                                  wsem.at[rb]).wait()

        for slot in range(NIB):
            stage_idx(slot, slot)
        wait_idx(0)
        start_gathers(0, 0)

        @pl.loop(0, nblk, step=4)
        def _(s):
            for b in range(4):
                j = s + b
                rb = b % 2

                @pl.when(j + 1 < nblk)
                def _():
                    wait_idx((b + 1) % 4)
                    start_gathers((b + 1) % 4, 1 - rb)

                wait_gathers(rb)

                @pl.when(j + 4 < nblk)
                def _():
                    stage_idx(j + 4, b)

                @pl.when(j >= 2)
                def _():
                    wait_writeout(rb)

                @pl.loop(0, BLK // 4)
                def _(q):
                    for u in range(4):
                        e = q * 4 + u
                        m = [ra_v[rb, e, pl.ds(c * LANES, LANES)] *
                             rb_v[rb, e, pl.ds(c * LANES, LANES)]
                             for c in range(D // LANES)]
                        while len(m) > 1:
                            m = [m[i] + m[i + 1]
                                 for i in range(0, len(m), 2)]
                        o_v[rb, e, :] = m[0]

                start_writeout(j, rb)

        wait_writeout(0)
        wait_writeout(1)

    return k(h, pair)


def _tc_linear(partials, W, b_row):
    _, N, D = partials.shape
    blk = 2048

    def body(p_ref, w_ref, b_ref, o_ref):
        s = p_ref[0] + p_ref[1]
        o_ref[...] = jnp.dot(s, w_ref[...],
                             preferred_element_type=jnp.float32) + b_ref[...]

    return pl.pallas_call(
        body,
        grid=(N // blk,),
        in_specs=[pl.BlockSpec((NC, blk, D), lambda i: (0, i, 0)),
                  pl.BlockSpec((D, D), lambda i: (0, 0)),
                  pl.BlockSpec((1, D), lambda i: (0, 0))],
        out_specs=pl.BlockSpec((blk, D), lambda i: (i, 0)),
        out_shape=jax.ShapeDtypeStruct((N, D), jnp.float32),
    )(partials, W, b_row)


def _tc_lanesum(parts):
    B, L = parts.shape
    blk = 4096

    def body(p_ref, o_ref):
        ones = jnp.ones((L, 1), jnp.float32)
        o_ref[...] = jnp.dot(p_ref[...], ones,
                             preferred_element_type=jnp.float32)

    return pl.pallas_call(
        body,
        grid=(B // blk,),
        in_specs=[pl.BlockSpec((blk, L), lambda i: (i, 0))],
        out_specs=pl.BlockSpec((blk, 1), lambda i: (i, 0)),
        out_shape=jax.ShapeDtypeStruct((B, 1), jnp.float32),
    )(parts)


def kernel(x, edge_index, edge_weight, edges, W, b):
    n_nodes, d = x.shape
    n_edges = edge_weight.shape[0]
    n_cand = edges.shape[1]

    unit = NW * BLK * 4
    e_pad = ((n_edges + unit - 1) // unit) * unit
    c_pad = ((n_cand + unit - 1) // unit) * unit
    nblk_e = e_pad // (NW * BLK)
    nblk_c = c_pad // (NW * BLK)

    src = _pad_to(edge_index[0].astype(jnp.int32), e_pad, 0)
    dst = _pad_to(edge_index[1].astype(jnp.int32), e_pad, 0)
    wts = _pad_to(edge_weight, e_pad, 0.0)
    e0 = _pad_to(edges[0].astype(jnp.int32), c_pad, 0)
    e1 = _pad_to(edges[1].astype(jnp.int32), c_pad, 0)

    epair = jnp.stack([src.reshape(NW, nblk_e, BLK),
                       dst.reshape(NW, nblk_e, BLK)], axis=2)
    w3 = wts.reshape(NW, nblk_e, 1, BLK)
    cpair = jnp.stack([e0.reshape(NW, nblk_c, BLK),
                       e1.reshape(NW, nblk_c, BLK)], axis=2)

    n_pad = ((n_nodes + NS * BLK - 1) // (NS * BLK)) * (NS * BLK)

    partials = _sc_gather_scale_scatter(x, epair, w3, n_pad)
    h = _tc_linear(partials, W, b.reshape(1, d))
    parts = _sc_pair_dot(h, cpair)
    out = _tc_lanesum(parts)
    return out[:n_cand, 0]
